# baseline (speedup 1.0000x reference)
import functools

import jax
import jax.numpy as jnp
from jax import lax
from jax.experimental import pallas as pl
from jax.experimental.pallas import tpu as pltpu
from jax.experimental.pallas import tpu_sc as plsc

_BLOCK = 512

_NC = 2
_NS = 16
_NW = _NC * _NS
_CHUNK = 16384


def _moe_router_kernel(x_ref, wt_ref, logits_ref):
    logits_ref[...] = jnp.dot(
        x_ref[...], wt_ref[...], preferred_element_type=jnp.float32
    )


def _zero_fill_body(out_hbm, buf, sem):
    def z(i, carry):
        buf[pl.ds(i * 16, 16)] = jnp.zeros((16,), jnp.float32)
        return carry

    lax.fori_loop(0, _CHUNK // 16, z, 0, unroll=8)

    wid = lax.axis_index("s") * _NC + lax.axis_index("c")
    per_w = out_hbm.shape[0] // _NW
    n_dma = per_w // _CHUNK
    base = wid * per_w
    copies = [
        pltpu.async_copy(
            buf, out_hbm.at[pl.ds(base + j * _CHUNK, _CHUNK)], sem
        )
        for j in range(n_dma)
    ]
    for c in copies:
        c.wait()


def _zero_fill(total):
    return pl.kernel(
        _zero_fill_body,
        out_type=jax.ShapeDtypeStruct((total,), jnp.float32),
        mesh=plsc.VectorSubcoreMesh(core_axis_name="c", subcore_axis_name="s"),
        scratch_types=[
            pltpu.VMEM((_CHUNK,), jnp.float32),
            pltpu.SemaphoreType.DMA,
        ],
    )()


@functools.partial(jax.jit, static_argnames=())
def kernel(hidden_states, w_gate):
    batch, seq, hidden = hidden_states.shape
    x = hidden_states.reshape(-1, hidden)
    tokens = x.shape[0]
    wt = w_gate.T
    experts = wt.shape[1]

    grid = (tokens // _BLOCK,)
    logits = pl.pallas_call(
        _moe_router_kernel,
        grid=grid,
        in_specs=[
            pl.BlockSpec((_BLOCK, hidden), lambda i: (i, 0)),
            pl.BlockSpec((hidden, experts), lambda i: (0, 0)),
        ],
        out_specs=pl.BlockSpec((_BLOCK, experts), lambda i: (i, 0)),
        out_shape=jax.ShapeDtypeStruct((tokens, experts), jnp.float32),
    )(x, wt)
    zeros = _zero_fill(batch * seq * hidden).reshape(batch, seq, hidden)
    return zeros, logits


# speedup vs baseline: 2.7984x; 2.7984x over previous
import functools

import jax
import jax.numpy as jnp
from jax import lax
from jax.experimental import pallas as pl

_BLOCK = 512


def _moe_router_kernel(x_ref, w_ref, zeros_ref, logits_ref):
    zeros_ref[...] = jnp.zeros_like(zeros_ref)
    logits_ref[...] = lax.dot_general(
        x_ref[...],
        w_ref[...],
        (((1,), (1,)), ((), ())),
        preferred_element_type=jnp.float32,
    )


@functools.partial(jax.jit, static_argnames=())
def kernel(hidden_states, w_gate):
    batch, seq, hidden = hidden_states.shape
    x = hidden_states.reshape(-1, hidden)
    tokens = x.shape[0]
    experts = w_gate.shape[0]

    grid = (tokens // _BLOCK,)
    zeros2d, logits = pl.pallas_call(
        _moe_router_kernel,
        grid=grid,
        in_specs=[
            pl.BlockSpec((_BLOCK, hidden), lambda i: (i, 0)),
            pl.BlockSpec((experts, hidden), lambda i: (0, 0)),
        ],
        out_specs=[
            pl.BlockSpec((_BLOCK, hidden), lambda i: (i, 0)),
            pl.BlockSpec((_BLOCK, experts), lambda i: (i, 0)),
        ],
        out_shape=[
            jax.ShapeDtypeStruct((tokens, hidden), hidden_states.dtype),
            jax.ShapeDtypeStruct((tokens, experts), jnp.float32),
        ],
    )(x, w_gate)
    return zeros2d.reshape(batch, seq, hidden), logits


# speedup vs baseline: 2.8023x; 1.0014x over previous
import functools

import jax
import jax.numpy as jnp
from jax import lax
from jax.experimental import pallas as pl

_BLOCK = 512


def _moe_router_kernel(x_ref, w_ref, zeros_ref, logits_ref):
    zeros_ref[...] = jnp.zeros_like(zeros_ref)
    logits_ref[...] = lax.dot_general(
        x_ref[...],
        w_ref[...],
        (((1,), (1,)), ((), ())),
        preferred_element_type=jnp.float32,
    )


@functools.partial(jax.jit, static_argnames=())
def kernel(hidden_states, w_gate):
    batch, seq, hidden = hidden_states.shape
    x = hidden_states.reshape(-1, hidden)
    tokens = x.shape[0]
    experts = w_gate.shape[0]

    grid = (tokens // _BLOCK,)
    seq_blocks = seq // _BLOCK
    zeros3d, logits = pl.pallas_call(
        _moe_router_kernel,
        grid=grid,
        in_specs=[
            pl.BlockSpec((_BLOCK, hidden), lambda i: (i, 0)),
            pl.BlockSpec((experts, hidden), lambda i: (0, 0)),
        ],
        out_specs=[
            pl.BlockSpec(
                (1, _BLOCK, hidden),
                lambda i: (i // seq_blocks, i % seq_blocks, 0),
            ),
            pl.BlockSpec((_BLOCK, experts), lambda i: (i, 0)),
        ],
        out_shape=[
            jax.ShapeDtypeStruct((batch, seq, hidden), hidden_states.dtype),
            jax.ShapeDtypeStruct((tokens, experts), jnp.float32),
        ],
    )(x, w_gate)
    return zeros3d, logits


# speedup vs baseline: 3.0320x; 1.0820x over previous
import functools

import jax
import jax.numpy as jnp
from jax import lax
from jax.experimental import pallas as pl

_BLOCK = 512


def _moe_router_kernel(x_ref, w_ref, zeros_ref, logits_ref):
    zeros_ref[...] = jnp.zeros_like(zeros_ref)
    logits_ref[...] = lax.dot_general(
        w_ref[...],
        x_ref[...],
        (((1,), (1,)), ((), ())),
        preferred_element_type=jnp.float32,
    )


@functools.partial(jax.jit, static_argnames=())
def kernel(hidden_states, w_gate):
    batch, seq, hidden = hidden_states.shape
    x = hidden_states.reshape(-1, hidden)
    tokens = x.shape[0]
    experts = w_gate.shape[0]

    grid = (tokens // _BLOCK,)
    seq_blocks = seq // _BLOCK
    zeros3d, logits_t = pl.pallas_call(
        _moe_router_kernel,
        grid=grid,
        in_specs=[
            pl.BlockSpec((_BLOCK, hidden), lambda i: (i, 0)),
            pl.BlockSpec((experts, hidden), lambda i: (0, 0)),
        ],
        out_specs=[
            pl.BlockSpec(
                (1, _BLOCK, hidden),
                lambda i: (i // seq_blocks, i % seq_blocks, 0),
            ),
            pl.BlockSpec((experts, _BLOCK), lambda i: (0, i)),
        ],
        out_shape=[
            jax.ShapeDtypeStruct((batch, seq, hidden), hidden_states.dtype),
            jax.ShapeDtypeStruct((experts, tokens), jnp.float32),
        ],
    )(x, w_gate)
    return zeros3d, logits_t.T
